# baseline (speedup 1.0000x reference)
import functools

import jax
import jax.numpy as jnp
from jax import lax
from jax.experimental import pallas as pl
from jax.experimental.pallas import tpu as pltpu
from jax.experimental.pallas import tpu_sc as plsc

L = 50
B = 16384
EMBED = 32
CH = 1024
NIDX = CH // 128
NCHUNK = B // CH
TPW = L // 2


def _sc_gather(idx2d, table):
    mesh = plsc.VectorSubcoreMesh(core_axis_name="c", subcore_axis_name="s")

    @functools.partial(
        pl.kernel,
        mesh=mesh,
        compiler_params=pltpu.CompilerParams(
            use_tc_tiling_on_sc=False, needs_layout_passes=False),
        out_type=jax.ShapeDtypeStruct((L, 4, B // 128, 8, 128), jnp.float32),
        scratch_types=[
            pltpu.VMEM((NIDX, 128), jnp.int32),
            pltpu.VMEM((NIDX, 128), jnp.int32),
            pltpu.VMEM((NIDX, 128, EMBED), jnp.float32),
            pltpu.VMEM((NIDX, 128, EMBED), jnp.float32),
            pltpu.VMEM((4, NIDX, 8, 128), jnp.float32),
            pltpu.SemaphoreType.DMA,
            pltpu.SemaphoreType.DMA,
            pltpu.SemaphoreType.DMA,
            pltpu.SemaphoreType.DMA,
            pltpu.SemaphoreType.DMA,
        ],
    )
    def k(idx_hbm, table_hbm, out_hbm,
          idx0, idx1, rows0, rows1, t4,
          si0, si1, sg0, sg1, sw):
        wid = lax.axis_index("s") * 2 + lax.axis_index("c")
        c = wid % NCHUNK
        l0 = wid // NCHUNK

        iota = lax.iota(jnp.int32, 16)
        row_vecs = [c0 + iota for c0 in range(0, 128, 16)]
        j_vecs = [jnp.full((16,), j, jnp.int32) for j in range(NIDX)]

        def load_idx(kk, idx_v, si):
            pltpu.async_copy(
                idx_hbm.at[pl.ds(128 * (l0 + 2 * kk) + NIDX * c, NIDX)],
                idx_v, si)

        def wait_idx(idx_v, si):
            pltpu.make_async_copy(idx_hbm.at[pl.ds(0, NIDX)], idx_v, si).wait()

        def fire(idx_v, rows, sg):
            for j in range(NIDX):
                pltpu.async_copy(table_hbm.at[idx_v.at[j]], rows.at[j], sg)

        def wait_gathers(idx_v, rows, sg):
            for j in range(NIDX):
                pltpu.make_async_copy(
                    table_hbm.at[idx_v.at[j]], rows.at[j], sg).wait()

        def retile(rows):
            @plsc.parallel_loop(0, EMBED, 1, unroll=4)
            def step(m):
                e_vec = jnp.bitwise_and(m + iota, EMBED - 1)
                e_hi = jnp.right_shift(e_vec, 3)
                e_lo = jnp.bitwise_and(e_vec, 7)
                for tcc in range(NIDX):
                    for ci in range(8):
                        v = plsc.load_gather(
                            rows, [j_vecs[tcc], row_vecs[ci], e_vec])
                        plsc.store_scatter(
                            t4, [e_hi, j_vecs[tcc], e_lo, row_vecs[ci]], v)

        def start_write(kk):
            pltpu.async_copy(
                t4, out_hbm.at[l0 + 2 * kk, :, pl.ds(NIDX * c, NIDX)], sw)

        def wait_write():
            pltpu.make_async_copy(
                t4, out_hbm.at[l0, :, pl.ds(NIDX * c, NIDX)], sw).wait()

        load_idx(0, idx0, si0)
        wait_idx(idx0, si0)
        fire(idx0, rows0, sg0)
        load_idx(1, idx1, si1)

        def half(kk, idx_p, si_p, rows_p, sg_p, idx_q, si_q, rows_q, sg_q,
                 first, last):
            @pl.when(jnp.logical_not(last))
            def _():
                wait_idx(idx_q, si_q)
                fire(idx_q, rows_q, sg_q)

            wait_gathers(idx_p, rows_p, sg_p)

            @pl.when(jnp.logical_and(jnp.logical_not(last), kk + 2 < TPW))
            def _():
                load_idx(kk + 2, idx_p, si_p)

            @pl.when(jnp.logical_not(first))
            def _():
                wait_write()

            retile(rows_p)
            start_write(kk)

        def body2(i, carry):
            k0 = 2 * i
            half(k0, idx0, si0, rows0, sg0, idx1, si1, rows1, sg1,
                 k0 == 0, False)
            half(k0 + 1, idx1, si1, rows1, sg1, idx0, si0, rows0, sg0,
                 False, k0 + 1 == TPW - 1)
            return carry

        lax.fori_loop(0, TPW // 2, body2, 0)
        half(TPW - 1, idx0, si0, rows0, sg0, idx1, si1, rows1, sg1,
             False, True)
        wait_write()

    return k(idx2d, table)


VB = 8192


def _tc_linearize(t_t):
    def body(x_ref, o_ref):
        ident = jnp.eye(EMBED, dtype=jnp.float32)
        y = lax.dot_general(
            x_ref[...], ident, (((0,), (0,)), ((), ())),
            precision=lax.Precision.HIGHEST,
            preferred_element_type=jnp.float32)
        y3 = y.reshape(VB // 4, 4, EMBED)
        for q in range(4):
            o_ref[:, q * EMBED:(q + 1) * EMBED] = y3[:, q, :]

    grid = (1000000 + VB - 1) // VB
    return pl.pallas_call(
        body,
        grid=(grid,),
        in_specs=[pl.BlockSpec((EMBED, VB), lambda i: (0, i))],
        out_specs=pl.BlockSpec((VB // 4, 128), lambda i: (i, 0)),
        out_shape=jax.ShapeDtypeStruct((250000, 128), jnp.float32),
    )(t_t)


def kernel(inputs, table):
    idx2d = inputs.T.reshape((B * L) // 128, 128)
    tlin = _tc_linearize(table.T).reshape(table.shape)
    out5d = _sc_gather(idx2d, tlin)
    return out5d.transpose(0, 2, 4, 1, 3).reshape(L, B, EMBED)


# speedup vs baseline: 1.6490x; 1.6490x over previous
import functools

import jax
import jax.numpy as jnp
from jax import lax
from jax.experimental import pallas as pl
from jax.experimental.pallas import tpu as pltpu
from jax.experimental.pallas import tpu_sc as plsc

L = 50
B = 16384
EMBED = 32
CH = 1024
NIDX = CH // 128
NCHUNK = B // CH
TPW = L // 2


def _sc_gather(idx2d, table):
    mesh = plsc.VectorSubcoreMesh(core_axis_name="c", subcore_axis_name="s")

    @functools.partial(
        pl.kernel,
        mesh=mesh,
        compiler_params=pltpu.CompilerParams(
            use_tc_tiling_on_sc=False, needs_layout_passes=False),
        out_type=jax.ShapeDtypeStruct((L, 4, B // 128, 8, 128), jnp.float32),
        scratch_types=[
            pltpu.VMEM((NIDX, 128), jnp.int32),
            pltpu.VMEM((NIDX, 128), jnp.int32),
            pltpu.VMEM((NIDX, 128, EMBED), jnp.float32),
            pltpu.VMEM((NIDX, 128, EMBED), jnp.float32),
            pltpu.VMEM((4, NIDX, 8, 128), jnp.float32),
            pltpu.SemaphoreType.DMA,
            pltpu.SemaphoreType.DMA,
            pltpu.SemaphoreType.DMA,
            pltpu.SemaphoreType.DMA,
            pltpu.SemaphoreType.DMA,
        ],
    )
    def k(idx_hbm, table_hbm, out_hbm,
          idx0, idx1, rows0, rows1, t4,
          si0, si1, sg0, sg1, sw):
        wid = lax.axis_index("s") * 2 + lax.axis_index("c")
        c = wid % NCHUNK
        l0 = wid // NCHUNK

        iota = lax.iota(jnp.int32, 16)
        row_vecs = [c0 + iota for c0 in range(0, 128, 16)]
        j_vecs = [jnp.full((16,), j, jnp.int32) for j in range(NIDX)]

        def load_idx(kk, idx_v, si):
            pltpu.async_copy(
                idx_hbm.at[pl.ds(128 * (l0 + 2 * kk) + NIDX * c, NIDX)],
                idx_v, si)

        def wait_idx(idx_v, si):
            pltpu.make_async_copy(idx_hbm.at[pl.ds(0, NIDX)], idx_v, si).wait()

        def fire(idx_v, rows, sg):
            for j in range(NIDX):
                pltpu.async_copy(table_hbm.at[idx_v.at[j]], rows.at[j], sg)

        def wait_gathers(idx_v, rows, sg):
            for j in range(NIDX):
                pltpu.make_async_copy(
                    table_hbm.at[idx_v.at[j]], rows.at[j], sg).wait()

        def retile(rows):
            @plsc.parallel_loop(0, EMBED, 1, unroll=4)
            def step(m):
                e_vec = jnp.bitwise_and(m + iota, EMBED - 1)
                e_hi = jnp.right_shift(e_vec, 3)
                e_lo = jnp.bitwise_and(e_vec, 7)
                for tcc in range(NIDX):
                    for ci in range(8):
                        v = plsc.load_gather(
                            rows, [j_vecs[tcc], row_vecs[ci], e_vec])
                        plsc.store_scatter(
                            t4, [e_hi, j_vecs[tcc], e_lo, row_vecs[ci]], v)

        def start_write(kk):
            pltpu.async_copy(
                t4, out_hbm.at[l0 + 2 * kk, :, pl.ds(NIDX * c, NIDX)], sw)

        def wait_write():
            pltpu.make_async_copy(
                t4, out_hbm.at[l0, :, pl.ds(NIDX * c, NIDX)], sw).wait()

        load_idx(0, idx0, si0)
        wait_idx(idx0, si0)
        fire(idx0, rows0, sg0)
        load_idx(1, idx1, si1)

        def half(kk, idx_p, si_p, rows_p, sg_p, idx_q, si_q, rows_q, sg_q,
                 first, last):
            @pl.when(jnp.logical_not(last))
            def _():
                wait_idx(idx_q, si_q)
                fire(idx_q, rows_q, sg_q)

            wait_gathers(idx_p, rows_p, sg_p)

            @pl.when(jnp.logical_and(jnp.logical_not(last), kk + 2 < TPW))
            def _():
                load_idx(kk + 2, idx_p, si_p)

            @pl.when(jnp.logical_not(first))
            def _():
                wait_write()

            retile(rows_p)
            start_write(kk)

        def body2(i, carry):
            k0 = 2 * i
            half(k0, idx0, si0, rows0, sg0, idx1, si1, rows1, sg1,
                 k0 == 0, False)
            half(k0 + 1, idx1, si1, rows1, sg1, idx0, si0, rows0, sg0,
                 False, k0 + 1 == TPW - 1)
            return carry

        lax.fori_loop(0, TPW // 2, body2, 0)
        half(TPW - 1, idx0, si0, rows0, sg0, idx1, si1, rows1, sg1,
             False, True)
        wait_write()

    return k(idx2d, table)


VB = 8192


def _tc_linearize(t_t):
    def body(x_ref, o_ref):
        y3 = x_ref[...].T.reshape(VB // 4, 4, EMBED)
        for q in range(4):
            o_ref[:, q * EMBED:(q + 1) * EMBED] = y3[:, q, :]

    grid = (1000000 + VB - 1) // VB
    return pl.pallas_call(
        body,
        grid=(grid,),
        in_specs=[pl.BlockSpec((EMBED, VB), lambda i: (0, i))],
        out_specs=pl.BlockSpec((VB // 4, 128), lambda i: (i, 0)),
        out_shape=jax.ShapeDtypeStruct((250000, 128), jnp.float32),
    )(t_t)


def kernel(inputs, table):
    idx2d = inputs.T.reshape((B * L) // 128, 128)
    tlin = _tc_linearize(table.T).reshape(table.shape)
    out5d = _sc_gather(idx2d, tlin)
    return out5d.transpose(0, 2, 4, 1, 3).reshape(L, B, EMBED)
